# baseline (speedup 1.0000x reference)
import jax
import jax.numpy as jnp
from jax import lax
from jax.experimental import pallas as pl
from jax.experimental.pallas import tpu as pltpu
from jax.experimental.pallas import tpu_sc as plsc

NUM_CORES = 2
NUM_SUBCORES = 16
NUM_WORKERS = NUM_CORES * NUM_SUBCORES
LANES = 16
NBUF = 4
PAD = 128


def _build(batch, seq, feat, scale):
    bpw = batch // NUM_WORKERS
    ftiles = feat // 8
    dims = (seq, ftiles, batch // bpw, 8, bpw)
    assert bpw == 128 and seq % NBUF == 0 and seq >= 3 * NBUF

    skew = 33

    def body(idx_hbm, table_hbm, out_hbm,
             idx_v, idx_t, gbuf, tbuf, gsems, ssems):
        wid = lax.axis_index("s") * NUM_CORES + lax.axis_index("c")
        iota = lax.iota(jnp.int32, LANES)

        pltpu.sync_copy(idx_hbm.at[pl.ds(wid * bpw, bpw)], idx_v)

        @pl.loop(0, seq)
        def _(s):
            svec = jnp.full((LANES,), s, jnp.int32)
            for b0 in range(0, bpw, LANES):
                v = plsc.load_gather(idx_v, [b0 + iota, svec])
                idx_t[s, pl.ds(b0, LANES)] = v * scale

        def fire_gather(s, k):
            pltpu.async_copy(table_hbm.at[idx_t.at[s]], gbuf.at[k], gsems.at[k])

        def wait_gather(k):
            pltpu.make_async_copy(
                table_hbm.at[pl.ds(0, bpw)], gbuf.at[k], gsems.at[k]
            ).wait()

        def transpose(k):
            for tf in range(ftiles):
                for fs in range(8):
                    fvec = jnp.full((LANES,), tf * 8 + fs, jnp.int32)
                    for b0 in range(0, bpw, LANES):
                        v = plsc.load_gather(gbuf.at[k], [b0 + iota, fvec])
                        tbuf[k, tf, fs, pl.ds(b0, LANES)] = v

        def fire_store(s, k):
            pltpu.async_copy(tbuf.at[k], out_hbm.at[s, :, wid], ssems.at[k])

        def wait_store(k):
            pltpu.make_async_copy(
                tbuf.at[k], out_hbm.at[0, :, 0], ssems.at[k]
            ).wait()

        for s in range(NBUF):
            fire_gather(s, s)
        for s in range(NBUF):
            wait_gather(s)
            transpose(s)
            fire_store(s, s)
            fire_gather(s + NBUF, s)

        @pl.loop(NBUF, seq - NBUF, step=NBUF)
        def _(t):
            for k in range(NBUF):
                s = t + k
                wait_gather(k)
                wait_store(k)
                transpose(k)
                fire_store(s, k)
                fire_gather(s + NBUF, k)

        for i in range(NBUF):
            s = seq - NBUF + i
            wait_gather(i)
            wait_store(i)
            transpose(i)
            fire_store(s, i)
        for k in range(NBUF):
            wait_store(k)

    return pl.kernel(
        body,
        out_type=jax.ShapeDtypeStruct(dims, jnp.float32),
        mesh=plsc.VectorSubcoreMesh(core_axis_name="c", subcore_axis_name="s"),
        scratch_types=[
            pltpu.VMEM((bpw, seq), jnp.int32),
            pltpu.VMEM((seq, bpw), jnp.int32),
            pltpu.VMEM((NBUF, bpw, feat), jnp.float32),
            pltpu.VMEM((NBUF, ftiles, 8, bpw), jnp.float32),
            pltpu.SemaphoreType.DMA((NBUF,)),
            pltpu.SemaphoreType.DMA((NBUF,)),
        ],
        compiler_params=pltpu.CompilerParams(use_tc_tiling_on_sc=False, needs_layout_passes=False),
    )


_PACK_BLK = 16384


def _pack_table(emb_t):
    feat, n = emb_t.shape

    def body(in_ref, out_ref):
        y = in_ref[...].T
        out_ref[...] = jnp.concatenate(
            [y, jnp.zeros((_PACK_BLK, PAD - feat), jnp.float32)], axis=1)

    return pl.pallas_call(
        body,
        grid=(pl.cdiv(n, _PACK_BLK),),
        in_specs=[pl.BlockSpec((feat, _PACK_BLK), lambda i: (0, i))],
        out_specs=pl.BlockSpec((_PACK_BLK, PAD), lambda i: (i, 0)),
        out_shape=jax.ShapeDtypeStruct((n, PAD), jnp.float32),
    )(emb_t)


def kernel(inputs, embedding):
    batch, seq = inputs.shape
    nemb, feat = embedding.shape
    scale = PAD // feat
    table = _pack_table(embedding.T).reshape(nemb * scale, feat)
    out5 = _build(batch, seq, feat, scale)(inputs, table)
    return out5.transpose(2, 4, 0, 1, 3).reshape(batch, seq, feat)


# speedup vs baseline: 1.0745x; 1.0745x over previous
import jax
import jax.numpy as jnp
from jax import lax
from jax.experimental import pallas as pl
from jax.experimental.pallas import tpu as pltpu
from jax.experimental.pallas import tpu_sc as plsc

NUM_CORES = 2
NUM_SUBCORES = 16
NUM_WORKERS = NUM_CORES * NUM_SUBCORES
LANES = 16
NBUF = 2
PAD = 128


def _build(batch, seq, feat, scale):
    bpw = batch // NUM_WORKERS
    ftiles = feat // 8
    dims = (seq, ftiles, batch // bpw, 8, bpw)
    assert bpw == 128 and seq % NBUF == 0 and seq >= 3 * NBUF

    skew = 33

    def body(idx_hbm, table_hbm, out_hbm,
             idx_v, idx_t, gbuf, tbuf, gsems, ssems):
        wid = lax.axis_index("s") * NUM_CORES + lax.axis_index("c")
        iota = lax.iota(jnp.int32, LANES)

        pltpu.sync_copy(idx_hbm.at[pl.ds(wid * bpw, bpw)], idx_v)

        @pl.loop(0, seq)
        def _(s):
            svec = jnp.full((LANES,), s, jnp.int32)
            for b0 in range(0, bpw, LANES):
                v = plsc.load_gather(idx_v, [b0 + iota, svec])
                idx_t[s, pl.ds(b0, LANES)] = v * scale

        def fire_gather(s, k):
            pltpu.async_copy(table_hbm.at[idx_t.at[s]], gbuf.at[k], gsems.at[k])

        def wait_gather(k):
            pltpu.make_async_copy(
                table_hbm.at[pl.ds(0, bpw)], gbuf.at[k], gsems.at[k]
            ).wait()

        def transpose(k):
            for tf in range(ftiles):
                for fs in range(8):
                    fvec = jnp.full((LANES,), tf * 8 + fs, jnp.int32)
                    for b0 in range(0, bpw, LANES):
                        v = plsc.load_gather(gbuf.at[k], [b0 + iota, fvec])
                        tbuf[k, tf, fs, pl.ds(b0, LANES)] = v

        def fire_store(s, k):
            pltpu.async_copy(tbuf.at[k], out_hbm.at[s, :, wid], ssems.at[k])

        def wait_store(k):
            pltpu.make_async_copy(
                tbuf.at[k], out_hbm.at[0, :, 0], ssems.at[k]
            ).wait()

        for s in range(NBUF):
            fire_gather(s, s)
        for s in range(NBUF):
            wait_gather(s)
            transpose(s)
            fire_store(s, s)
            fire_gather(s + NBUF, s)

        @pl.loop(NBUF, seq - NBUF, step=NBUF)
        def _(t):
            for k in range(NBUF):
                s = t + k
                wait_gather(k)
                wait_store(k)
                transpose(k)
                fire_store(s, k)
                fire_gather(s + NBUF, k)

        for i in range(NBUF):
            s = seq - NBUF + i
            wait_gather(i)
            wait_store(i)
            transpose(i)
            fire_store(s, i)
        for k in range(NBUF):
            wait_store(k)

    return pl.kernel(
        body,
        out_type=jax.ShapeDtypeStruct(dims, jnp.float32),
        mesh=plsc.VectorSubcoreMesh(core_axis_name="c", subcore_axis_name="s"),
        scratch_types=[
            pltpu.VMEM((bpw, seq), jnp.int32),
            pltpu.VMEM((seq, bpw), jnp.int32),
            pltpu.VMEM((NBUF, bpw, feat), jnp.float32),
            pltpu.VMEM((NBUF, ftiles, 8, bpw), jnp.float32),
            pltpu.SemaphoreType.DMA((NBUF,)),
            pltpu.SemaphoreType.DMA((NBUF,)),
        ],
        compiler_params=pltpu.CompilerParams(use_tc_tiling_on_sc=False, needs_layout_passes=False),
    )


_PACK_BLK = 16384


def _pack_table(emb_t):
    feat, n = emb_t.shape

    def body(in_ref, out_ref):
        y = in_ref[...].T
        out_ref[...] = jnp.concatenate(
            [y, jnp.zeros((_PACK_BLK, PAD - feat), jnp.float32)], axis=1)

    return pl.pallas_call(
        body,
        grid=(pl.cdiv(n, _PACK_BLK),),
        in_specs=[pl.BlockSpec((feat, _PACK_BLK), lambda i: (0, i))],
        out_specs=pl.BlockSpec((_PACK_BLK, PAD), lambda i: (i, 0)),
        out_shape=jax.ShapeDtypeStruct((n, PAD), jnp.float32),
    )(emb_t)


def kernel(inputs, embedding):
    batch, seq = inputs.shape
    nemb, feat = embedding.shape
    scale = PAD // feat
    table = _pack_table(embedding.T).reshape(nemb * scale, feat)
    out5 = _build(batch, seq, feat, scale)(inputs, table)
    return out5.transpose(2, 4, 0, 1, 3).reshape(batch, seq, feat)


# speedup vs baseline: 1.6618x; 1.5465x over previous
import jax
import jax.numpy as jnp
from jax import lax
from jax.experimental import pallas as pl
from jax.experimental.pallas import tpu as pltpu
from jax.experimental.pallas import tpu_sc as plsc

NUM_CORES = 2
NUM_SUBCORES = 16
NUM_WORKERS = NUM_CORES * NUM_SUBCORES
LANES = 16
NBUF = 4
PAD = 128


def _build(batch, seq, feat, scale):
    bpw = batch // NUM_WORKERS
    ftiles = feat // 8
    dims = (seq, batch // bpw, bpw, PAD)
    assert bpw == 128 and seq % NBUF == 0 and seq >= 3 * NBUF

    skew = 33

    def body(idx_hbm, table_hbm, out_hbm,
             idx_v, idx_t, gbuf, gsems, ssems):
        wid = lax.axis_index("s") * NUM_CORES + lax.axis_index("c")
        iota = lax.iota(jnp.int32, LANES)

        pltpu.sync_copy(idx_hbm.at[pl.ds(wid * bpw, bpw)], idx_v)

        @pl.loop(0, seq)
        def _(s):
            svec = jnp.full((LANES,), s, jnp.int32)
            for b0 in range(0, bpw, LANES):
                v = plsc.load_gather(idx_v, [b0 + iota, svec])
                idx_t[s, pl.ds(b0, LANES)] = v * scale

        def fire_gather(s, k):
            pltpu.async_copy(table_hbm.at[idx_t.at[s]], gbuf.at[k], gsems.at[k])

        def wait_gather(k):
            pltpu.make_async_copy(
                table_hbm.at[pl.ds(0, bpw)], gbuf.at[k], gsems.at[k]
            ).wait()

        def fire_store(s, k):
            pltpu.async_copy(
                gbuf.at[k], out_hbm.at[s, wid, :, pl.ds(0, feat)], ssems.at[k])

        def wait_store(k):
            pltpu.make_async_copy(
                gbuf.at[k], out_hbm.at[0, 0, :, pl.ds(0, feat)], ssems.at[k]
            ).wait()

        for k in range(NBUF):
            fire_gather(k, k)
        for k in range(NBUF):
            wait_gather(k)
            fire_store(k, k)

        @pl.loop(NBUF, seq, step=NBUF)
        def _(t):
            for k in range(NBUF):
                wait_store(k)
                fire_gather(t + k, k)
            for k in range(NBUF):
                wait_gather(k)
                fire_store(t + k, k)

        for k in range(NBUF):
            wait_store(k)

    return pl.kernel(
        body,
        out_type=jax.ShapeDtypeStruct(dims, jnp.float32),
        mesh=plsc.VectorSubcoreMesh(core_axis_name="c", subcore_axis_name="s"),
        scratch_types=[
            pltpu.VMEM((bpw, seq), jnp.int32),
            pltpu.VMEM((seq, bpw), jnp.int32),
            pltpu.VMEM((NBUF, bpw, feat), jnp.float32),
            pltpu.SemaphoreType.DMA((NBUF,)),
            pltpu.SemaphoreType.DMA((NBUF,)),
        ],
        compiler_params=pltpu.CompilerParams(use_tc_tiling_on_sc=False, needs_layout_passes=False),
    )


_PACK_BLK = 16384


def _pack_table(emb_t):
    feat, n = emb_t.shape

    def body(in_ref, out_ref):
        y = in_ref[...].T
        out_ref[...] = jnp.concatenate(
            [y, jnp.zeros((_PACK_BLK, PAD - feat), jnp.float32)], axis=1)

    return pl.pallas_call(
        body,
        grid=(pl.cdiv(n, _PACK_BLK),),
        in_specs=[pl.BlockSpec((feat, _PACK_BLK), lambda i: (0, i))],
        out_specs=pl.BlockSpec((_PACK_BLK, PAD), lambda i: (i, 0)),
        out_shape=jax.ShapeDtypeStruct((n, PAD), jnp.float32),
    )(emb_t)


def _unpack_out(raw, feat):
    seq, batch, _ = raw.shape

    def body(in_ref, out_ref):
        out_ref[0] = in_ref[0][:, :feat].T

    return pl.pallas_call(
        body,
        grid=(seq,),
        in_specs=[pl.BlockSpec((1, batch, PAD), lambda i: (i, 0, 0))],
        out_specs=pl.BlockSpec((1, feat, batch), lambda i: (i, 0, 0)),
        out_shape=jax.ShapeDtypeStruct((seq, feat, batch), jnp.float32),
    )(raw)


def kernel(inputs, embedding):
    batch, seq = inputs.shape
    nemb, feat = embedding.shape
    scale = PAD // feat
    table = _pack_table(embedding.T).reshape(nemb * scale, feat)
    raw = _build(batch, seq, feat, scale)(inputs, table)
    sfb = _unpack_out(raw.reshape(seq, batch, PAD), feat)
    return sfb.transpose(2, 0, 1)
